# baseline (speedup 1.0000x reference)
import math

import jax
import jax.numpy as jnp
from jax.experimental import pallas as pl

_V = 100000
_B = 1024
_H = _B // 2
_SMOOTH = 0.1 / (_V - 2)
_CONF = 0.9
_ENT = (_V - 2) * _SMOOTH * math.log(_SMOOTH) + _CONF * math.log(_CONF)
_BLK = 2048
_GRID = (_V + _BLK - 1) // _BLK


def _half(d, t, col):
    m = t != 0
    w = _SMOOTH * (col != 0).astype(jnp.float32) \
        + (_CONF - _SMOOTH) * (col == t).astype(jnp.float32)
    part = jnp.sum(jnp.where(col < _V, w * jnp.where(m, d, 0.0), 0.0))
    nv = jnp.sum(jnp.where(m, 1.0, 0.0))
    return part, nv


def _body(ta_ref, tb_ref, a_ref, b_ref, acc_ref):
    j = pl.program_id(0)
    col = j * _BLK + jax.lax.broadcasted_iota(jnp.int32, (_H, _BLK), 1)
    pa, nva = _half(a_ref[...], ta_ref[...], col)
    pb, nvb = _half(b_ref[...], tb_ref[...], col)

    @pl.when(j == 0)
    def _():
        acc_ref[...] = jnp.full((1, 1), _ENT, jnp.float32) * (nva + nvb)

    acc_ref[...] -= pa + pb


def kernel(output, target):
    t2 = target.reshape(_B, 1)
    acc = pl.pallas_call(
        _body,
        grid=(_GRID,),
        in_specs=[
            pl.BlockSpec((_H, 1), lambda j: (0, 0)),
            pl.BlockSpec((_H, 1), lambda j: (1, 0)),
            pl.BlockSpec((_H, _BLK), lambda j: (0, j)),
            pl.BlockSpec((_H, _BLK), lambda j: (1, j)),
        ],
        out_specs=pl.BlockSpec((1, 1), lambda j: (0, 0)),
        out_shape=jax.ShapeDtypeStruct((1, 1), jnp.float32),
    )(t2, t2, output, output)
    return acc[0, 0]


# speedup vs baseline: 1.0009x; 1.0009x over previous
import math

import jax
import jax.numpy as jnp
from jax.experimental import pallas as pl

_V = 100000
_B = 1024
_SMOOTH = 0.1 / (_V - 2)
_CONF = 0.9
_ENT = (_V - 2) * _SMOOTH * math.log(_SMOOTH) + _CONF * math.log(_CONF)
_BLK = 2048
_GRID = (_V + _BLK - 1) // _BLK


def _body(tgt_ref, out_ref, acc_ref):
    j = pl.program_id(0)
    d = out_ref[...]
    t = tgt_ref[...]
    m = (t != 0).astype(jnp.float32)
    col = j * _BLK + jax.lax.broadcasted_iota(jnp.int32, (_B, _BLK), 1)
    w = _SMOOTH * (col != 0).astype(jnp.float32) \
        + (_CONF - _SMOOTH) * (col == t).astype(jnp.float32)
    part = jnp.sum(jnp.where(col < _V, w * m * d, 0.0))

    @pl.when(j == 0)
    def _():
        acc_ref[...] = jnp.full((1, 1), _ENT, jnp.float32) * jnp.sum(m)

    acc_ref[...] -= part


def kernel(output, target):
    t2 = target.reshape(_B, 1)
    acc = pl.pallas_call(
        _body,
        grid=(_GRID,),
        in_specs=[
            pl.BlockSpec((_B, 1), lambda j: (0, 0)),
            pl.BlockSpec((_B, _BLK), lambda j: (0, j)),
        ],
        out_specs=pl.BlockSpec((1, 1), lambda j: (0, 0)),
        out_shape=jax.ShapeDtypeStruct((1, 1), jnp.float32),
    )(t2, output)
    return acc[0, 0]


# speedup vs baseline: 1.0230x; 1.0221x over previous
import math

import jax
import jax.numpy as jnp
from jax.experimental import pallas as pl

_V = 100000
_B = 1024
_SMOOTH = 0.1 / (_V - 2)
_CONF = 0.9
_ENT = (_V - 2) * _SMOOTH * math.log(_SMOOTH) + _CONF * math.log(_CONF)
_BLK = 4096
_GRID = (_V + _BLK - 1) // _BLK


def _body(tgt_ref, out_ref, acc_ref):
    j = pl.program_id(0)
    d = out_ref[...]
    t = tgt_ref[...]
    m = (t != 0).astype(jnp.float32)
    col = j * _BLK + jax.lax.broadcasted_iota(jnp.int32, (_B, _BLK), 1)
    w = _SMOOTH * (col != 0).astype(jnp.float32) \
        + (_CONF - _SMOOTH) * (col == t).astype(jnp.float32)
    part = jnp.sum(jnp.where(col < _V, w * m * d, 0.0))

    @pl.when(j == 0)
    def _():
        acc_ref[...] = jnp.full((1, 1), _ENT, jnp.float32) * jnp.sum(m)

    acc_ref[...] -= part


def kernel(output, target):
    t2 = target.reshape(_B, 1)
    acc = pl.pallas_call(
        _body,
        grid=(_GRID,),
        in_specs=[
            pl.BlockSpec((_B, 1), lambda j: (0, 0)),
            pl.BlockSpec((_B, _BLK), lambda j: (0, j)),
        ],
        out_specs=pl.BlockSpec((1, 1), lambda j: (0, 0)),
        out_shape=jax.ShapeDtypeStruct((1, 1), jnp.float32),
    )(t2, output)
    return acc[0, 0]
